# Initial kernel scaffold; baseline (speedup 1.0000x reference)
import jax
import jax.numpy as jnp
from jax.experimental import pallas as pl


def kernel(z, embedding_weight):
    raise NotImplementedError("write your pallas kernel here")


# speedup vs baseline: 1.1399x; 1.1399x over previous
import functools

import jax
import jax.numpy as jnp
from jax import lax
from jax.experimental import pallas as pl
from jax.experimental.pallas import tpu as pltpu
from jax.experimental.pallas import tpu_sc as plsc

_BETA = 0.25
_K = 8192
_D = 256
_N = 8192
_TBLK = 512
_KBLK = 2048


def _dist_body(z_ref, e_ref, idx_ref, loss_ref):
    i = pl.program_id(0)
    z = z_ref[...]
    sumz = jnp.sum(z * z, axis=1, keepdims=True)

    run_min = None
    run_idx = None
    for kb in range(_K // _KBLK):
        e = e_ref[pl.ds(kb * _KBLK, _KBLK), :]
        sume = jnp.sum(e * e, axis=1)
        mm = lax.dot_general(
            z, e, (((1,), (1,)), ((), ())),
            preferred_element_type=jnp.float32,
        )
        d = sumz - 2.0 * mm + sume[None, :]
        m = jnp.min(d, axis=1, keepdims=True)
        iota = lax.broadcasted_iota(jnp.int32, d.shape, 1)
        bidx = jnp.min(jnp.where(d == m, iota, _K), axis=1, keepdims=True)
        bidx = bidx + (kb * _KBLK)
        if run_min is None:
            run_min, run_idx = m, bidx
        else:
            better = m < run_min
            run_idx = jnp.where(better, bidx, run_idx)
            run_min = jnp.where(better, m, run_min)

    idx_ref[0, 0, :] = run_idx[:, 0]
    part = jnp.sum(run_min)

    @pl.when(i == 0)
    def _():
        loss_ref[0, 0] = part

    @pl.when(i > 0)
    def _():
        loss_ref[0, 0] = loss_ref[0, 0] + part


def _distances_argmin(z_flat, emb):
    grid = (_N // _TBLK,)
    idx, losssum = pl.pallas_call(
        _dist_body,
        grid=grid,
        in_specs=[
            pl.BlockSpec((_TBLK, _D), lambda i: (i, 0)),
            pl.BlockSpec((_K, _D), lambda i: (0, 0)),
        ],
        out_specs=[
            pl.BlockSpec((1, 1, _TBLK), lambda i: (i, 0, 0)),
            pl.BlockSpec(memory_space=pltpu.SMEM),
        ],
        out_shape=[
            jax.ShapeDtypeStruct((_N // _TBLK, 1, _TBLK), jnp.int32),
            jax.ShapeDtypeStruct((1, 1), jnp.float32),
        ],
    )(z_flat, emb)
    return idx, losssum


_NC, _NS = 2, 16
_NW = _NC * _NS
_CH = 128
_NROWS = _N // _CH
_CPW = _NROWS // _NW


@functools.cache
def _make_sc_gather():
    @functools.partial(
        pl.kernel,
        out_type=jax.ShapeDtypeStruct((_NROWS, _CH, _D), jnp.float32),
        mesh=plsc.VectorSubcoreMesh(core_axis_name="c", subcore_axis_name="s",
                                    num_cores=_NC, num_subcores=_NS),
        scratch_types=[
            pltpu.VMEM((_CPW, _CH), jnp.int32),
            pltpu.VMEM((_CPW, _CH, _D), jnp.float32),
            pltpu.SemaphoreType.DMA,
        ],
    )
    def _sc_gather(table_hbm, idx_hbm, out_hbm, idx_v, rows_v, sem):
        wid = lax.axis_index("s") * _NC + lax.axis_index("c")
        base = wid * _CPW
        pltpu.sync_copy(idx_hbm.at[pl.ds(base, _CPW)], idx_v)
        for j in range(_CPW):
            pltpu.async_copy(table_hbm.at[idx_v.at[j]], rows_v.at[j], sem).wait()
        pltpu.sync_copy(rows_v, out_hbm.at[pl.ds(base, _CPW)])

    return _sc_gather


def kernel(z, embedding_weight):
    B, D, H, W = z.shape
    z_flat = jnp.transpose(z, (0, 2, 3, 1)).reshape(-1, D)

    idx3d, losssum = _distances_argmin(z_flat, embedding_weight)
    indices = idx3d.reshape(B, H, W)

    zq_rows = _make_sc_gather()(embedding_weight, idx3d.reshape(_NROWS, _CH))
    z_q = jnp.transpose(zq_rows.reshape(B, H, W, D), (0, 3, 1, 2))

    loss = (losssum[0, 0] / jnp.float32(_N * _D)) * jnp.float32(_BETA)
    return (z_q, indices, loss)


# speedup vs baseline: 1.4088x; 1.2358x over previous
import functools

import jax
import jax.numpy as jnp
from jax import lax
from jax.experimental import pallas as pl
from jax.experimental.pallas import tpu as pltpu
from jax.experimental.pallas import tpu_sc as plsc

_BETA = 0.25
_K = 8192
_D = 256
_N = 8192
_TBLK = 512
_KBLK = 2048


def _dist_body(z_ref, e_ref, idx_ref, loss_ref, sume_ref):
    i = pl.program_id(0)
    z = z_ref[...]

    @pl.when(i == 0)
    def _():
        for kb in range(_K // _KBLK):
            e = e_ref[pl.ds(kb * _KBLK, _KBLK), :]
            sume_ref[0, pl.ds(kb * _KBLK, _KBLK)] = jnp.sum(e * e, axis=1)

    sumz = jnp.sum(z * z, axis=1, keepdims=True)

    run_min = None
    run_idx = None
    for kb in range(_K // _KBLK):
        e = e_ref[pl.ds(kb * _KBLK, _KBLK), :]
        sume = sume_ref[0, pl.ds(kb * _KBLK, _KBLK)]
        mm = lax.dot_general(
            z, e, (((1,), (1,)), ((), ())),
            preferred_element_type=jnp.float32,
        )
        d = sumz - 2.0 * mm + sume[None, :]
        m = jnp.min(d, axis=1, keepdims=True)
        iota = lax.broadcasted_iota(jnp.int32, (1, _KBLK), 1).astype(jnp.float32)
        bidx = jnp.min(jnp.where(d == m, iota, jnp.float32(_K)),
                       axis=1, keepdims=True)
        bidx = bidx + jnp.float32(kb * _KBLK)
        if run_min is None:
            run_min, run_idx = m, bidx
        else:
            better = m < run_min
            run_idx = jnp.where(better, bidx, run_idx)
            run_min = jnp.where(better, m, run_min)

    idx_ref[0, 0, :] = run_idx[:, 0].astype(jnp.int32)
    part = jnp.sum(run_min)

    @pl.when(i == 0)
    def _():
        loss_ref[0, 0] = part

    @pl.when(i > 0)
    def _():
        loss_ref[0, 0] = loss_ref[0, 0] + part


def _distances_argmin(z_flat, emb):
    grid = (_N // _TBLK,)
    idx, losssum = pl.pallas_call(
        _dist_body,
        grid=grid,
        in_specs=[
            pl.BlockSpec((_TBLK, _D), lambda i: (i, 0)),
            pl.BlockSpec((_K, _D), lambda i: (0, 0)),
        ],
        out_specs=[
            pl.BlockSpec((1, 1, _TBLK), lambda i: (i, 0, 0)),
            pl.BlockSpec(memory_space=pltpu.SMEM),
        ],
        out_shape=[
            jax.ShapeDtypeStruct((_N // _TBLK, 1, _TBLK), jnp.int32),
            jax.ShapeDtypeStruct((1, 1), jnp.float32),
        ],
        scratch_shapes=[pltpu.VMEM((1, _K), jnp.float32)],
    )(z_flat, emb)
    return idx, losssum


_NC, _NS = 2, 16
_NW = _NC * _NS
_CH = 128
_NROWS = _N // _CH
_CPW = _NROWS // _NW


@functools.cache
def _make_sc_gather():
    @functools.partial(
        pl.kernel,
        out_type=jax.ShapeDtypeStruct((_NROWS, _CH, _D), jnp.float32),
        mesh=plsc.VectorSubcoreMesh(core_axis_name="c", subcore_axis_name="s",
                                    num_cores=_NC, num_subcores=_NS),
        scratch_types=[
            pltpu.VMEM((_CPW, _CH), jnp.int32),
            pltpu.VMEM((_CPW, _CH, _D), jnp.float32),
            pltpu.SemaphoreType.DMA,
        ],
    )
    def _sc_gather(table_hbm, idx_hbm, out_hbm, idx_v, rows_v, sem):
        wid = lax.axis_index("s") * _NC + lax.axis_index("c")
        base = wid * _CPW
        pltpu.sync_copy(idx_hbm.at[pl.ds(base, _CPW)], idx_v)
        for j in range(_CPW):
            pltpu.async_copy(table_hbm.at[idx_v.at[j]], rows_v.at[j], sem).wait()
        pltpu.sync_copy(rows_v, out_hbm.at[pl.ds(base, _CPW)])

    return _sc_gather


def kernel(z, embedding_weight):
    B, D, H, W = z.shape
    z_flat = jnp.transpose(z, (0, 2, 3, 1)).reshape(-1, D)

    idx3d, losssum = _distances_argmin(z_flat, embedding_weight)
    indices = idx3d.reshape(B, H, W)

    zq_rows = _make_sc_gather()(embedding_weight, idx3d.reshape(_NROWS, _CH))
    z_q = jnp.transpose(zq_rows.reshape(B, H, W, D), (0, 3, 1, 2))

    loss = (losssum[0, 0] / jnp.float32(_N * _D)) * jnp.float32(_BETA)
    return (z_q, indices, loss)
